# baseline (speedup 1.0000x reference)
import functools

import jax
import jax.numpy as jnp
from jax import lax
from jax.experimental import pallas as pl
from jax.experimental.pallas import tpu as pltpu
from jax.experimental.pallas import tpu_sc as plsc

B = 4
S = 2048
H = 1024
N = B * S
NC = 2
NS = 16
NW = NC * NS
ROWS_PER_W = N // NW
K = 16
NCHUNK = ROWS_PER_W // K
NV = H // 16
EPS = 1e-7


def _lane_sum(x):
    lanes = lax.iota(jnp.int32, 16)
    for d in (8, 4, 2, 1):
        x = x + x.at[lanes ^ d].get(mode="promise_in_bounds", unique_indices=True)
    return x


def _rsqrt(v):
    i = lax.bitcast_convert_type(v, jnp.int32)
    y = lax.bitcast_convert_type(jnp.int32(0x5F3759DF) - (i >> 1), jnp.float32)
    for _ in range(3):
        y = y * (1.5 - 0.5 * v * y * y)
    return y


def _sc_embed(ids, mask, word_table, pos_table, gamma, beta):
    mesh = plsc.VectorSubcoreMesh(core_axis_name="c", subcore_axis_name="s")

    @functools.partial(
        pl.kernel,
        mesh=mesh,
        out_type=jax.ShapeDtypeStruct((N, H), jnp.float32),
        scratch_types=[
            pltpu.VMEM((ROWS_PER_W,), jnp.int32),
            pltpu.VMEM((K, H), jnp.float32),
            pltpu.VMEM((K, H), jnp.float32),
            pltpu.VMEM((K,), jnp.float32),
            pltpu.VMEM((H,), jnp.float32),
            pltpu.VMEM((H,), jnp.float32),
            pltpu.SemaphoreType.DMA,
        ],
    )
    def k(ids_hbm, mask_hbm, word_hbm, pos_hbm, gamma_hbm, beta_hbm, out_hbm,
          idx_v, x_v, pos_v, mask_v, gamma_v, beta_v, sem):
        wid = lax.axis_index("s") * NC + lax.axis_index("c")
        base = wid * ROWS_PER_W
        pos_base = lax.rem(base, S)

        pltpu.sync_copy(ids_hbm.at[pl.ds(base, ROWS_PER_W)], idx_v)
        pltpu.sync_copy(gamma_hbm, gamma_v)
        pltpu.sync_copy(beta_hbm, beta_v)

        def chunk_body(c, carry):
            off = c * K
            pltpu.async_copy(
                word_hbm.at[idx_v.at[pl.ds(off, K)]], x_v, sem).wait()
            pltpu.sync_copy(pos_hbm.at[pl.ds(pos_base + off, K)], pos_v)
            pltpu.sync_copy(mask_hbm.at[pl.ds(base + off, K)], mask_v)

            m_all = mask_v[...]

            def row_body(r, _):
                def sum_body(j, sc):
                    s1, s2 = sc
                    t = x_v[r, pl.ds(j * 16, 16)] + pos_v[r, pl.ds(j * 16, 16)]
                    x_v[r, pl.ds(j * 16, 16)] = t
                    return (s1 + t, s2 + t * t)

                z = jnp.zeros((16,), jnp.float32)
                s1, s2 = lax.fori_loop(0, NV, sum_body, (z, z))
                mu_v = _lane_sum(s1) * (1.0 / H)
                var = _lane_sum(s2) * (1.0 / H) - mu_v * mu_v
                a = _rsqrt(var + EPS)
                m = m_all.at[jnp.full((16,), r, jnp.int32)].get(
                    mode="promise_in_bounds")
                am = a * m

                def norm_body(j, _):
                    t = x_v[r, pl.ds(j * 16, 16)]
                    g = gamma_v[pl.ds(j * 16, 16)]
                    bb = beta_v[pl.ds(j * 16, 16)]
                    x_v[r, pl.ds(j * 16, 16)] = (t - mu_v) * am * g + bb * m
                    return 0

                lax.fori_loop(0, NV, norm_body, 0)
                return 0

            lax.fori_loop(0, K, row_body, 0)
            pltpu.sync_copy(x_v, out_hbm.at[pl.ds(base + off, K)])
            return carry

        lax.fori_loop(0, NCHUNK, chunk_body, 0)

    return k(ids, mask, word_table, pos_table, gamma, beta)


def kernel(input_ids, mask, word_table, pos_table, gamma, beta):
    ids = input_ids.reshape(N).astype(jnp.int32)
    mask_flat = mask.reshape(N).astype(jnp.float32)
    out = _sc_embed(ids, mask_flat, word_table, pos_table, gamma, beta)
    return out.reshape(B, S, H)
